# Initial kernel scaffold; baseline (speedup 1.0000x reference)
import jax
import jax.numpy as jnp
from jax.experimental import pallas as pl


def kernel(sampling_map, refined_response_maps, selected_regions):
    raise NotImplementedError("write your pallas kernel here")


# speedup vs baseline: 8.9378x; 8.9378x over previous
import functools

import jax
import jax.numpy as jnp
from jax import lax
from jax.experimental import pallas as pl
from jax.experimental.pallas import tpu as pltpu
from jax.experimental.pallas import tpu_sc as plsc

_GRID = 4
_WIN = 3
_NC = 2
_NS = 16
_NW = _NC * _NS


def _fuser_body(tiles_per_worker, gh, gw, n_row_tiles, n_col_tiles,
                smp_hbm, ref_hbm, sel_hbm, out_hbm, selv, buf):
    cid = lax.axis_index("c")
    sid = lax.axis_index("s")
    wid = sid * _NC + cid
    base = wid * tiles_per_worker
    pltpu.sync_copy(sel_hbm.at[pl.ds(wid * 16, 16)], selv)
    sv = selv[...]

    tiles_per_b = n_row_tiles * n_col_tiles

    for t in range(tiles_per_worker):
        tid = base + t
        b = tid // tiles_per_b
        rem = tid % tiles_per_b
        ti = rem // n_col_tiles
        tj = rem % n_col_tiles
        y = ti * gh
        x = tj * gw
        kk = sv[t]

        @pl.when(kk >= 0)
        def _():
            pltpu.sync_copy(ref_hbm.at[b, kk, pl.ds(y, gh), pl.ds(x, gw)],
                            buf)

        @pl.when(kk < 0)
        def _():
            pltpu.sync_copy(smp_hbm.at[b, pl.ds(y, gh), pl.ds(x, gw)], buf)

        def sig_row(r, c2):
            for c in range(gw // 16):
                v = buf[r, pl.ds(c * 16, 16)]
                buf[r, pl.ds(c * 16, 16)] = 1.0 / (1.0 + jnp.exp(-v))
            return c2

        lax.fori_loop(0, gh, sig_row, 0)
        pltpu.sync_copy(buf, out_hbm.at[b, pl.ds(y, gh), pl.ds(x, gw)])


def kernel(sampling_map, refined_response_maps, selected_regions):
    B, C, H, W = sampling_map.shape
    K = refined_response_maps.shape[1]
    gh = H // _GRID
    gw = W // _GRID

    rb = jnp.clip(selected_regions[:, :, 0], 0, (H - _WIN * gh) // gh)
    cb = jnp.clip(selected_regions[:, :, 1], 0, (W - _WIN * gw) // gw)
    ti = jnp.arange(_GRID)[None, None, :, None]
    tj = jnp.arange(_GRID)[None, None, None, :]
    rbx = rb[:, :, None, None]
    cbx = cb[:, :, None, None]
    cov = ((ti >= rbx) & (ti < rbx + _WIN)
           & (tj >= cbx) & (tj < cbx + _WIN))
    ks = jnp.arange(K, dtype=jnp.int32)[None, :, None, None]
    kstar = jnp.max(jnp.where(cov, ks, jnp.int32(-1)), axis=1)
    n_tiles = B * _GRID * _GRID
    tiles_per_worker = n_tiles // _NW
    sel = jnp.zeros((_NW, 16), jnp.int32)
    sel = sel.at[:, :tiles_per_worker].set(
        kstar.reshape(_NW, tiles_per_worker)).reshape(_NW * 16)

    smp = sampling_map.reshape(B, H, W)
    ref = refined_response_maps.reshape(B, K, H, W)

    mesh = plsc.VectorSubcoreMesh(core_axis_name="c", subcore_axis_name="s")
    body = functools.partial(_fuser_body, tiles_per_worker, gh, gw,
                             _GRID, _GRID)
    fn = pl.kernel(
        body,
        out_type=jax.ShapeDtypeStruct((B, H, W), jnp.float32),
        mesh=mesh,
        scratch_types=[
            pltpu.VMEM((16,), jnp.int32),
            pltpu.VMEM((gh, gw), jnp.float32),
        ],
    )
    out = fn(smp, ref, sel)
    return out.reshape(B, C, H, W)


# speedup vs baseline: 13.7549x; 1.5390x over previous
import functools

import jax
import jax.numpy as jnp
from jax import lax
from jax.experimental import pallas as pl
from jax.experimental.pallas import tpu as pltpu
from jax.experimental.pallas import tpu_sc as plsc

_GRID = 4
_WIN = 3
_NC = 2
_NS = 16
_NW = _NC * _NS
_NBUF = 2


def _fuser_body(tiles_per_worker, gh, gw, n_row_tiles, n_col_tiles,
                smp_hbm, ref_hbm, sel_hbm, out_hbm, selv,
                ibuf0, ibuf1, obuf0, obuf1,
                isem0, isem1, osem0, osem1):
    ibufs = (ibuf0, ibuf1)
    obufs = (obuf0, obuf1)
    isems = (isem0, isem1)
    osems = (osem0, osem1)

    cid = lax.axis_index("c")
    sid = lax.axis_index("s")
    wid = sid * _NC + cid
    base = wid * tiles_per_worker
    pltpu.sync_copy(sel_hbm.at[pl.ds(wid * 16, 16)], selv)
    sv = selv[...]

    tiles_per_b = n_row_tiles * n_col_tiles

    def tile_slice(t):
        tid = base + t
        b = tid // tiles_per_b
        rem = tid % tiles_per_b
        y = (rem // n_col_tiles) * gh
        x = (rem % n_col_tiles) * gw
        return b, y, x

    def start_in(t):
        b, y, x = tile_slice(t)
        kk = sv[t]
        buf = ibufs[t % _NBUF]
        sem = isems[t % _NBUF]

        @pl.when(kk >= 0)
        def _():
            pltpu.async_copy(
                ref_hbm.at[b, kk, pl.ds(y, gh), pl.ds(x, gw)], buf, sem)

        @pl.when(kk < 0)
        def _():
            pltpu.async_copy(
                smp_hbm.at[b, pl.ds(y, gh), pl.ds(x, gw)], buf, sem)

    def wait_in(t):
        pltpu.make_async_copy(
            smp_hbm.at[0, pl.ds(0, gh), pl.ds(0, gw)],
            ibufs[t % _NBUF], isems[t % _NBUF]).wait()

    def start_out(t):
        b, y, x = tile_slice(t)
        pltpu.async_copy(
            obufs[t % _NBUF],
            out_hbm.at[b, pl.ds(y, gh), pl.ds(x, gw)], osems[t % _NBUF])

    def wait_out(t):
        b, y, x = tile_slice(t)
        pltpu.make_async_copy(
            obufs[t % _NBUF],
            out_hbm.at[b, pl.ds(y, gh), pl.ds(x, gw)], osems[t % _NBUF]).wait()

    def sigmoid_tile(src, dst):
        def row(r, c2):
            for c in range(gw // 16):
                v = src[r, pl.ds(c * 16, 16)]
                dst[r, pl.ds(c * 16, 16)] = 1.0 / (1.0 + jnp.exp(-v))
            return c2

        lax.fori_loop(0, gh, row, 0)

    for t in range(min(_NBUF, tiles_per_worker)):
        start_in(t)
    for t in range(tiles_per_worker):
        wait_in(t)
        if t >= _NBUF:
            wait_out(t - _NBUF)
        sigmoid_tile(ibufs[t % _NBUF], obufs[t % _NBUF])
        start_out(t)
        if t + _NBUF < tiles_per_worker:
            start_in(t + _NBUF)
    for t in range(max(0, tiles_per_worker - _NBUF), tiles_per_worker):
        wait_out(t)


def kernel(sampling_map, refined_response_maps, selected_regions):
    B, C, H, W = sampling_map.shape
    K = refined_response_maps.shape[1]
    gh = H // _GRID
    gw = W // _GRID

    rb = jnp.clip(selected_regions[:, :, 0], 0, (H - _WIN * gh) // gh)
    cb = jnp.clip(selected_regions[:, :, 1], 0, (W - _WIN * gw) // gw)
    ti = jnp.arange(_GRID)[None, None, :, None]
    tj = jnp.arange(_GRID)[None, None, None, :]
    rbx = rb[:, :, None, None]
    cbx = cb[:, :, None, None]
    cov = ((ti >= rbx) & (ti < rbx + _WIN)
           & (tj >= cbx) & (tj < cbx + _WIN))
    ks = jnp.arange(K, dtype=jnp.int32)[None, :, None, None]
    kstar = jnp.max(jnp.where(cov, ks, jnp.int32(-1)), axis=1)
    n_tiles = B * _GRID * _GRID
    tiles_per_worker = n_tiles // _NW
    sel = jnp.zeros((_NW, 16), jnp.int32)
    sel = sel.at[:, :tiles_per_worker].set(
        kstar.reshape(_NW, tiles_per_worker)).reshape(_NW * 16)

    smp = sampling_map.reshape(B, H, W)
    ref = refined_response_maps.reshape(B, K, H, W)

    mesh = plsc.VectorSubcoreMesh(core_axis_name="c", subcore_axis_name="s")
    body = functools.partial(_fuser_body, tiles_per_worker, gh, gw,
                             _GRID, _GRID)
    fn = pl.kernel(
        body,
        out_type=jax.ShapeDtypeStruct((B, H, W), jnp.float32),
        mesh=mesh,
        scratch_types=[
            pltpu.VMEM((16,), jnp.int32),
            pltpu.VMEM((gh, gw), jnp.float32),
            pltpu.VMEM((gh, gw), jnp.float32),
            pltpu.VMEM((gh, gw), jnp.float32),
            pltpu.VMEM((gh, gw), jnp.float32),
            pltpu.SemaphoreType.DMA,
            pltpu.SemaphoreType.DMA,
            pltpu.SemaphoreType.DMA,
            pltpu.SemaphoreType.DMA,
        ],
    )
    out = fn(smp, ref, sel)
    return out.reshape(B, C, H, W)


# speedup vs baseline: 14.2042x; 1.0327x over previous
import functools

import jax
import jax.numpy as jnp
from jax import lax
from jax.experimental import pallas as pl
from jax.experimental.pallas import tpu as pltpu
from jax.experimental.pallas import tpu_sc as plsc

_GRID = 4
_WIN = 3
_NC = 2
_NS = 16
_NW = _NC * _NS
_NBUF = 3
_ROWS_PER_ITER = 2


def _fuser_body(tiles_per_worker, gh, gw, n_row_tiles, n_col_tiles,
                smp_hbm, ref_hbm, sel_hbm, out_hbm, selv, *rest):
    ibufs = rest[0:_NBUF]
    obufs = rest[_NBUF:2 * _NBUF]
    isems = rest[2 * _NBUF:3 * _NBUF]
    osems = rest[3 * _NBUF:4 * _NBUF]

    cid = lax.axis_index("c")
    sid = lax.axis_index("s")
    wid = sid * _NC + cid
    base = wid * tiles_per_worker
    pltpu.sync_copy(sel_hbm.at[pl.ds(wid * 16, 16)], selv)
    sv = selv[...]

    tiles_per_b = n_row_tiles * n_col_tiles

    def tile_slice(t):
        tid = base + t
        b = tid // tiles_per_b
        rem = tid % tiles_per_b
        y = (rem // n_col_tiles) * gh
        x = (rem % n_col_tiles) * gw
        return b, y, x

    def start_in(t):
        b, y, x = tile_slice(t)
        kk = sv[t]
        buf = ibufs[t % _NBUF]
        sem = isems[t % _NBUF]

        @pl.when(kk >= 0)
        def _():
            pltpu.async_copy(
                ref_hbm.at[b, kk, pl.ds(y, gh), pl.ds(x, gw)], buf, sem)

        @pl.when(kk < 0)
        def _():
            pltpu.async_copy(
                smp_hbm.at[b, pl.ds(y, gh), pl.ds(x, gw)], buf, sem)

    def wait_in(t):
        pltpu.make_async_copy(
            smp_hbm.at[0, pl.ds(0, gh), pl.ds(0, gw)],
            ibufs[t % _NBUF], isems[t % _NBUF]).wait()

    def start_out(t):
        b, y, x = tile_slice(t)
        pltpu.async_copy(
            obufs[t % _NBUF],
            out_hbm.at[b, pl.ds(y, gh), pl.ds(x, gw)], osems[t % _NBUF])

    def wait_out(t):
        b, y, x = tile_slice(t)
        pltpu.make_async_copy(
            obufs[t % _NBUF],
            out_hbm.at[b, pl.ds(y, gh), pl.ds(x, gw)], osems[t % _NBUF]).wait()

    def sigmoid_tile(src, dst):
        def rows(r2, c2):
            r0 = r2 * _ROWS_PER_ITER
            for dr in range(_ROWS_PER_ITER):
                for c in range(gw // 16):
                    v = src[r0 + dr, pl.ds(c * 16, 16)]
                    dst[r0 + dr, pl.ds(c * 16, 16)] = \
                        1.0 / (1.0 + jnp.exp(-v))
            return c2

        lax.fori_loop(0, gh // _ROWS_PER_ITER, rows, 0)

    for t in range(min(_NBUF, tiles_per_worker)):
        start_in(t)
    for t in range(tiles_per_worker):
        wait_in(t)
        if t >= _NBUF:
            wait_out(t - _NBUF)
        sigmoid_tile(ibufs[t % _NBUF], obufs[t % _NBUF])
        start_out(t)
        if t + _NBUF < tiles_per_worker:
            start_in(t + _NBUF)
    for t in range(max(0, tiles_per_worker - _NBUF), tiles_per_worker):
        wait_out(t)


def kernel(sampling_map, refined_response_maps, selected_regions):
    B, C, H, W = sampling_map.shape
    K = refined_response_maps.shape[1]
    gh = H // _GRID
    gw = W // _GRID

    rb = jnp.clip(selected_regions[:, :, 0], 0, (H - _WIN * gh) // gh)
    cb = jnp.clip(selected_regions[:, :, 1], 0, (W - _WIN * gw) // gw)
    ti = jnp.arange(_GRID)[None, None, :, None]
    tj = jnp.arange(_GRID)[None, None, None, :]
    rbx = rb[:, :, None, None]
    cbx = cb[:, :, None, None]
    cov = ((ti >= rbx) & (ti < rbx + _WIN)
           & (tj >= cbx) & (tj < cbx + _WIN))
    ks = jnp.arange(K, dtype=jnp.int32)[None, :, None, None]
    kstar = jnp.max(jnp.where(cov, ks, jnp.int32(-1)), axis=1)
    n_tiles = B * _GRID * _GRID
    tiles_per_worker = n_tiles // _NW
    sel = jnp.zeros((_NW, 16), jnp.int32)
    sel = sel.at[:, :tiles_per_worker].set(
        kstar.reshape(_NW, tiles_per_worker)).reshape(_NW * 16)

    smp = sampling_map.reshape(B, H, W)
    ref = refined_response_maps.reshape(B, K, H, W)

    mesh = plsc.VectorSubcoreMesh(core_axis_name="c", subcore_axis_name="s")
    body = functools.partial(_fuser_body, tiles_per_worker, gh, gw,
                             _GRID, _GRID)
    fn = pl.kernel(
        body,
        out_type=jax.ShapeDtypeStruct((B, H, W), jnp.float32),
        mesh=mesh,
        scratch_types=(
            [pltpu.VMEM((16,), jnp.int32)]
            + [pltpu.VMEM((gh, gw), jnp.float32)] * (2 * _NBUF)
            + [pltpu.SemaphoreType.DMA] * (2 * _NBUF)
        ),
    )
    out = fn(smp, ref, sel)
    return out.reshape(B, C, H, W)


# speedup vs baseline: 14.2660x; 1.0044x over previous
import functools

import jax
import jax.numpy as jnp
from jax import lax
from jax.experimental import pallas as pl
from jax.experimental.pallas import tpu as pltpu
from jax.experimental.pallas import tpu_sc as plsc

_GRID = 4
_WIN = 3
_NC = 2
_NS = 16
_NW = _NC * _NS
_NBUF = 3
_ROWS_PER_ITER = 2


def _fuser_body(tiles_per_worker, gh, gw, n_row_tiles, n_col_tiles, K,
                smp_hbm, ref_hbm, sr_hbm, out_hbm, srv, *rest):
    ibufs = rest[0:_NBUF]
    obufs = rest[_NBUF:2 * _NBUF]
    isems = rest[2 * _NBUF:3 * _NBUF]
    osems = rest[3 * _NBUF:4 * _NBUF]

    cid = lax.axis_index("c")
    sid = lax.axis_index("s")
    wid = sid * _NC + cid
    base = wid * tiles_per_worker
    tiles_per_b = n_row_tiles * n_col_tiles
    b = base // tiles_per_b
    rem0 = base % tiles_per_b

    pltpu.sync_copy(sr_hbm.at[pl.ds(b * 2 * K, 16)], srv)
    sv = srv[...]
    max_r = n_row_tiles - _WIN
    max_c = n_col_tiles - _WIN
    rks = [jnp.clip(sv[2 * k], 0, max_r) for k in range(K)]
    cks = [jnp.clip(sv[2 * k + 1], 0, max_c) for k in range(K)]

    def tile_state(t):
        rem = rem0 + t
        ti = rem // n_col_tiles
        tj = rem % n_col_tiles
        kk = jnp.int32(-1)
        for k in range(K):
            cov = ((ti >= rks[k]) & (ti < rks[k] + _WIN)
                   & (tj >= cks[k]) & (tj < cks[k] + _WIN))
            kk = jnp.where(cov, jnp.int32(k), kk)
        return kk, ti * gh, tj * gw

    def start_in(t):
        kk, y, x = tile_state(t)
        buf = ibufs[t % _NBUF]
        sem = isems[t % _NBUF]

        @pl.when(kk >= 0)
        def _():
            pltpu.async_copy(
                ref_hbm.at[b, kk, pl.ds(y, gh), pl.ds(x, gw)], buf, sem)

        @pl.when(kk < 0)
        def _():
            pltpu.async_copy(
                smp_hbm.at[b, pl.ds(y, gh), pl.ds(x, gw)], buf, sem)

    def wait_in(t):
        pltpu.make_async_copy(
            smp_hbm.at[0, pl.ds(0, gh), pl.ds(0, gw)],
            ibufs[t % _NBUF], isems[t % _NBUF]).wait()

    def start_out(t):
        _, y, x = tile_state(t)
        pltpu.async_copy(
            obufs[t % _NBUF],
            out_hbm.at[b, pl.ds(y, gh), pl.ds(x, gw)], osems[t % _NBUF])

    def wait_out(t):
        _, y, x = tile_state(t)
        pltpu.make_async_copy(
            obufs[t % _NBUF],
            out_hbm.at[b, pl.ds(y, gh), pl.ds(x, gw)], osems[t % _NBUF]).wait()

    def sigmoid_tile(src, dst):
        def rows(r2, c2):
            r0 = r2 * _ROWS_PER_ITER
            for dr in range(_ROWS_PER_ITER):
                for c in range(gw // 16):
                    v = src[r0 + dr, pl.ds(c * 16, 16)]
                    dst[r0 + dr, pl.ds(c * 16, 16)] = \
                        1.0 / (1.0 + jnp.exp(-v))
            return c2

        lax.fori_loop(0, gh // _ROWS_PER_ITER, rows, 0)

    for t in range(min(_NBUF, tiles_per_worker)):
        start_in(t)
    for t in range(tiles_per_worker):
        wait_in(t)
        if t >= _NBUF:
            wait_out(t - _NBUF)
        sigmoid_tile(ibufs[t % _NBUF], obufs[t % _NBUF])
        start_out(t)
        if t + _NBUF < tiles_per_worker:
            start_in(t + _NBUF)
    for t in range(max(0, tiles_per_worker - _NBUF), tiles_per_worker):
        wait_out(t)


def kernel(sampling_map, refined_response_maps, selected_regions):
    B, C, H, W = sampling_map.shape
    K = refined_response_maps.shape[1]
    gh = H // _GRID
    gw = W // _GRID

    sr = jnp.pad(selected_regions.reshape(B * K * 2), (0, 16))

    smp = sampling_map.reshape(B, H, W)
    ref = refined_response_maps.reshape(B, K, H, W)

    n_tiles = B * _GRID * _GRID
    tiles_per_worker = n_tiles // _NW

    mesh = plsc.VectorSubcoreMesh(core_axis_name="c", subcore_axis_name="s")
    body = functools.partial(_fuser_body, tiles_per_worker, gh, gw,
                             _GRID, _GRID, K)
    fn = pl.kernel(
        body,
        out_type=jax.ShapeDtypeStruct((B, H, W), jnp.float32),
        mesh=mesh,
        scratch_types=(
            [pltpu.VMEM((16,), jnp.int32)]
            + [pltpu.VMEM((gh, gw), jnp.float32)] * (2 * _NBUF)
            + [pltpu.SemaphoreType.DMA] * (2 * _NBUF)
        ),
    )
    out = fn(smp, ref, sr)
    return out.reshape(B, C, H, W)
